# Initial kernel scaffold; baseline (speedup 1.0000x reference)
import jax
import jax.numpy as jnp
from jax.experimental import pallas as pl


def kernel(w1a, b1a, w2a, b2a, w1c, b1c, w2c, b2c, SR, HDs, acts, values):
    raise NotImplementedError("write your pallas kernel here")


# speedup vs baseline: 1.4182x; 1.4182x over previous
import jax
import jax.numpy as jnp
from jax.experimental import pallas as pl
from jax.experimental.pallas import tpu as pltpu

LANES = 128


def _make_body(T, S, A):
    f32 = jnp.float32

    def body(sr_ref, hds_ref, acts_ref, vals_ref,
             w1s_ref, wh_ref, b1_ref, w2_ref, b2_ref,
             out_ref, val_ref):
        w1s = w1s_ref[...]
        b1 = b1_ref[...]
        w2 = w2_ref[...]
        b2 = b2_ref[...]
        wh = [wh_ref[i:i + 1] for i in range(3)]

        outs = []
        for t in range(T):
            x = sr_ref[:, S * t:S * (t + 1)]
            hp = jnp.dot(x, w1s, preferred_element_type=f32)
            hp = (hp
                  + hds_ref[:, t:t + 1] * wh[0]
                  + acts_ref[:, t:t + 1] * wh[1]
                  + vals_ref[:, t:t + 1] * wh[2]
                  + b1)
            h = jnp.tanh(hp)
            outs.append(jnp.dot(h, w2, preferred_element_type=f32) + b2)

        val_ref[...] = outs[0][:, A:A + 1]

        m = outs[0]
        for t in range(1, T):
            m = jnp.maximum(m, outs[t])
        se = jnp.exp(outs[0] - m)
        for t in range(1, T):
            se = se + jnp.exp(outs[t] - m)
        lse = m + jnp.log(se)
        for t in range(T):
            out_ref[:, A * t:A * (t + 1)] = (outs[t] - lse)[:, :A]

    return body


def _pick_b_block(B):
    for cand in (512, 256, 128, 64, 32, 16, 8):
        if B % cand == 0 and (B // cand) >= 2:
            return cand
    return B


def kernel(w1a, b1a, w2a, b2a, w1c, b1c, w2c, b2c, SR, HDs, acts, values):
    f32 = jnp.float32
    B, T, S = SR.shape
    H = w1a.shape[1]
    H2 = 2 * H
    A = w2a.shape[1]

    w1f = jnp.concatenate([w1a, w1c], axis=1).astype(f32)
    w1s = w1f[:S]
    wh = w1f[S:S + 3]
    b1f = jnp.concatenate([b1a, b1c], axis=1).astype(f32)
    w2f = jnp.zeros((H2, LANES), f32)
    w2f = w2f.at[:H, :A].set(w2a.astype(f32))
    w2f = w2f.at[H:, A:A + 1].set(w2c.astype(f32))
    b2f = jnp.zeros((1, LANES), f32)
    b2f = b2f.at[:, :A].set(b2a.astype(f32))
    b2f = b2f.at[:, A:A + 1].set(b2c.astype(f32))

    SRf = SR.reshape(B, T * S).astype(f32)
    HDsf = HDs.astype(f32)
    actsf = acts.astype(f32)
    valsf = values.astype(f32)

    b_block = _pick_b_block(B)
    nb = B // b_block
    row_tile = lambda i: (i, 0)
    full = lambda i: (0, 0)

    out_flat, val = pl.pallas_call(
        _make_body(T, S, A),
        grid=(nb,),
        in_specs=[
            pl.BlockSpec((b_block, T * S), row_tile),
            pl.BlockSpec((b_block, T), row_tile),
            pl.BlockSpec((b_block, T), row_tile),
            pl.BlockSpec((b_block, T), row_tile),
            pl.BlockSpec((S, H2), full),
            pl.BlockSpec((3, H2), full),
            pl.BlockSpec((1, H2), full),
            pl.BlockSpec((H2, LANES), full),
            pl.BlockSpec((1, LANES), full),
        ],
        out_specs=[
            pl.BlockSpec((b_block, T * A), row_tile),
            pl.BlockSpec((b_block, 1), row_tile),
        ],
        out_shape=[
            jax.ShapeDtypeStruct((B, T * A), f32),
            jax.ShapeDtypeStruct((B, 1), f32),
        ],
        compiler_params=pltpu.CompilerParams(
            dimension_semantics=("parallel",)),
    )(SRf, HDsf, actsf, valsf, w1s, wh, b1f, w2f, b2f)

    logp = out_flat.reshape(B, T, A)
    value = val.reshape(B)
    return logp, value
